# baseline (speedup 1.0000x reference)
import functools

import jax
import jax.numpy as jnp
from jax import lax
from jax.experimental import pallas as pl
from jax.experimental.pallas import tpu as pltpu
from jax.experimental.pallas import tpu_sc as plsc

N = 10000
D = 128
K = 8
KP = 16
E = 320000

NC = 2
NS = 16
NW = NC * NS
EPW = E // NW
CHUNK = 1000
MCH = EPW // CHUNK
NP = 10240
RPS = NP // NS


def _stage_a_body(x_ref, mt_ref, w_ref, v_ref, q_ref):
    u = lax.dot_general(w_ref[...], v_ref[...], (((2,), (1,)), ((0,), (0,))),
                        preferred_element_type=jnp.float32)
    up = jnp.concatenate([u, jnp.zeros((KP - K, D), jnp.float32)], axis=0)
    q = lax.dot_general(x_ref[...], up, (((1,), (1,)), ((), ())),
                        preferred_element_type=jnp.float32)
    m = lax.transpose(mt_ref[...], (1, 0))
    mp = jnp.concatenate([m, jnp.zeros((N, KP - K), jnp.float32)], axis=1)
    q_ref[...] = jnp.concatenate(
        [q * mp, jnp.zeros((NP - N, KP), jnp.float32)], axis=0)


_stage_a = pl.pallas_call(
    _stage_a_body,
    out_shape=jax.ShapeDtypeStruct((NP, KP), jnp.float32),
)


def _sc_body(q_hbm, ei_hbm, zq_hbm, out_hbm, idxa, idxb, rowsa, rowsb,
             stage_v, acc_sh, qtab_sh, semia, semib, semga, semgb):
    c = lax.axis_index("c")
    s = lax.axis_index("s")
    wid = s * NC + c

    row0 = pl.multiple_of(s * RPS, RPS)
    pltpu.sync_copy(q_hbm.at[pl.ds(row0, RPS)], stage_v)
    pltpu.sync_copy(stage_v, qtab_sh.at[pl.ds(row0, RPS)])
    pltpu.sync_copy(zq_hbm.at[pl.ds(row0, RPS)], stage_v)
    pltpu.sync_copy(stage_v, acc_sh.at[pl.ds(row0, RPS)])
    plsc.subcore_barrier()

    eb0 = pl.multiple_of(wid * EPW, 8)

    def i_copy(ci, idx_v, sem):
        off = pl.multiple_of(eb0 + ci * CHUNK, 8)
        return pltpu.make_async_copy(ei_hbm.at[:, pl.ds(off, CHUNK)],
                                     idx_v, sem)

    def g_copy(idx_v, rows_v, sem):
        return pltpu.make_async_copy(qtab_sh.at[idx_v.at[0]], rows_v, sem)

    def s_add(idx_v, rows_v):
        pltpu.sync_copy(rows_v, acc_sh.at[idx_v.at[1]], add=True)

    i_copy(0, idxa, semia).start()
    i_copy(1, idxb, semib).start()
    i_copy(0, idxa, semia).wait()
    g_copy(idxa, rowsa, semga).start()

    def body(t, carry):
        a = 2 * t
        i_copy(a + 1, idxb, semib).wait()
        g_copy(idxb, rowsb, semgb).start()
        g_copy(idxa, rowsa, semga).wait()
        s_add(idxa, rowsa)
        i_copy(a + 2, idxa, semia).start()
        g_copy(idxb, rowsb, semgb).wait()
        s_add(idxb, rowsb)
        i_copy(a + 3, idxb, semib).start()
        i_copy(a + 2, idxa, semia).wait()
        g_copy(idxa, rowsa, semga).start()
        return carry

    lax.fori_loop(0, MCH // 2 - 1, body, 0, unroll=False)

    i_copy(MCH - 1, idxb, semib).wait()
    g_copy(idxb, rowsb, semgb).start()
    g_copy(idxa, rowsa, semga).wait()
    s_add(idxa, rowsa)
    g_copy(idxb, rowsb, semgb).wait()
    s_add(idxb, rowsb)

    plsc.subcore_barrier()
    pltpu.sync_copy(acc_sh.at[pl.ds(row0, RPS)], stage_v)
    pltpu.sync_copy(stage_v, out_hbm.at[c, pl.ds(row0, RPS)])


@functools.cache
def _sc_segsum():
    return pl.kernel(
        _sc_body,
        out_type=jax.ShapeDtypeStruct((NC, NP, KP), jnp.float32),
        mesh=plsc.VectorSubcoreMesh(core_axis_name="c", subcore_axis_name="s",
                                    num_cores=NC, num_subcores=NS),
        scratch_types=[
            pltpu.VMEM((2, CHUNK), jnp.int32),
            pltpu.VMEM((2, CHUNK), jnp.int32),
            pltpu.VMEM((CHUNK, KP), jnp.float32),
            pltpu.VMEM((CHUNK, KP), jnp.float32),
            pltpu.VMEM((RPS, KP), jnp.float32),
            pltpu.VMEM_SHARED((NP, KP), jnp.float32),
            pltpu.VMEM_SHARED((NP, KP), jnp.float32),
            pltpu.SemaphoreType.DMA,
            pltpu.SemaphoreType.DMA,
            pltpu.SemaphoreType.DMA,
            pltpu.SemaphoreType.DMA,
        ],
        compiler_params=pltpu.CompilerParams(use_tc_tiling_on_sc=False),
    )


def _stage_b_body(x_ref, mt_ref, zp_ref, b_ref, out_ref):
    z = zp_ref[0, :N] + zp_ref[1, :N]
    w = jax.nn.sigmoid(z)
    m = lax.transpose(mt_ref[...], (1, 0))
    mp = jnp.concatenate([m, jnp.zeros((N, KP - K), jnp.float32)], axis=1)
    cw = mp * w
    xb = x_ref[...]
    num = lax.dot_general(xb, cw, (((0,), (0,)), ((), ())),
                          preferred_element_type=jnp.float32)
    ws = jnp.sum(w, axis=0, keepdims=True)
    ctr = num / (ws + 1e-8)
    cn2 = jnp.sum(ctr * ctr, axis=0, keepdims=True)
    xc = jnp.dot(xb, ctr, preferred_element_type=jnp.float32)
    sq = jnp.sum(xb * xb, axis=1, keepdims=True)
    bp = jnp.concatenate(
        [b_ref[...], jnp.full((1, KP - K), -jnp.inf, jnp.float32)], axis=1)
    score = 2.0 * xc - sq - cn2 + bp
    h = score > 0.0
    hf = h.astype(jnp.float32)
    ii = lax.broadcasted_iota(jnp.int32, (KP, KP), 0)
    jj = lax.broadcasted_iota(jnp.int32, (KP, KP), 1)
    tri = (ii <= jj).astype(jnp.float32)
    cnt = jnp.dot(hf, tri, preferred_element_type=jnp.float32)
    res = jnp.where(h & (cnt <= 2.0), 1.0, 0.0)
    out_ref[...] = lax.transpose(res[:, :K], (1, 0))


_stage_b = pl.pallas_call(
    _stage_b_body,
    out_shape=jax.ShapeDtypeStruct((K, N), jnp.float32),
)


def kernel(x, edge_index, mask, W, v, b):
    zq = jnp.zeros((NP, KP), jnp.float32)
    mt = mask.T
    q = _stage_a(x, mt, W, v)
    zparts = _sc_segsum()(q, edge_index, zq)
    outt = _stage_b(x, mt, zparts, b.reshape(1, K))
    return outt.T
